# baseline (speedup 1.0000x reference)
import functools

import jax
import jax.numpy as jnp
from jax.experimental import pallas as pl
from jax.experimental.pallas import tpu as pltpu

_BLK = 512


def _gat_kernel(x_ref, w_ref, a_ref, adj_ref, xb_ref, out_ref):
    hid = jnp.dot(x_ref[...], w_ref[...], preferred_element_type=jnp.float32)
    d = w_ref.shape[1]
    a1 = a_ref[:d, :]
    a2 = a_ref[d:, :]
    wa1 = jnp.dot(w_ref[...], a1, preferred_element_type=jnp.float32)
    s1_blk = jnp.dot(xb_ref[...], wa1, preferred_element_type=jnp.float32)
    s2 = jax.lax.dot_general(a2, hid, (((0,), (1,)), ((), ())),
                             preferred_element_type=jnp.float32)
    logits = s1_blk + s2
    neg = jnp.where(logits >= 0.0, logits, 0.2 * logits)
    e = jnp.where(adj_ref[...] != 0, jnp.exp(-neg), 0.0)
    rowsum = jnp.sum(e, axis=1, keepdims=True)
    h = jnp.dot(e, hid, preferred_element_type=jnp.float32)
    hp = h / (rowsum + 1e-9)
    out_ref[...] = jnp.where(hp > 0.0, hp, jnp.exp(jnp.minimum(hp, 0.0)) - 1.0)


@jax.jit
def kernel(x, adj, W, a):
    n, din = x.shape
    dout = W.shape[1]
    grid = n // _BLK
    return pl.pallas_call(
        _gat_kernel,
        grid=(grid,),
        in_specs=[
            pl.BlockSpec((n, din), lambda i: (0, 0)),
            pl.BlockSpec((din, dout), lambda i: (0, 0)),
            pl.BlockSpec((2 * dout, 1), lambda i: (0, 0)),
            pl.BlockSpec((_BLK, n), lambda i: (i, 0)),
            pl.BlockSpec((_BLK, din), lambda i: (i, 0)),
        ],
        out_specs=pl.BlockSpec((_BLK, dout), lambda i: (i, 0)),
        out_shape=jax.ShapeDtypeStruct((n, dout), jnp.float32),
        compiler_params=pltpu.CompilerParams(
            dimension_semantics=("parallel",)),
    )(x, W, a, adj, x)
